# Initial kernel scaffold; baseline (speedup 1.0000x reference)
import jax
import jax.numpy as jnp
from jax.experimental import pallas as pl


def kernel(me_c1_w, me_c1_b, me_c2_w, me_c2_b, mh_henc_w, mh_henc_b, mh_hdec_w, mh_hdec_b, mh_fsr, md_c1_w, md_c1_b, md_c2_w, md_c2_b, mcn_c1_w, mcn_c1_b, mcn_c2_w, mcn_c2_b, mcn_c3_w, mcn_c3_b, re_c1_w, re_c1_b, re_c2_w, re_c2_b, rh_henc_w, rh_henc_b, rh_hdec_w, rh_hdec_b, rh_fsr, rd_c1_w, rd_c1_b, rd_c2_w, rd_c2_b, frame1, frame2, flow12):
    raise NotImplementedError("write your pallas kernel here")


# speedup vs baseline: 2.0511x; 2.0511x over previous
import functools
import numpy as np

import jax
import jax.numpy as jnp
from jax.experimental import pallas as pl
from jax.experimental.pallas import tpu as pltpu

_INV_LN2 = float(1.0 / np.log(2.0))


def _ceil_to(x, m):
    return ((x + m - 1) // m) * m


def _apply_act(acc, act):
    if act == "relu":
        return jnp.maximum(acc, 0.0)
    if act == "softplus":
        return jnp.maximum(acc, 0.0) + jnp.log1p(jnp.exp(-jnp.abs(acc)))
    return acc


def _conv3_kernel(x_ref, w_ref, b_ref, o_ref, *, tm, w_pix, c_in, act):
    acc = None
    for dy in range(3):
        for dx in range(3):
            sl = x_ref[0, dy:dy + tm, dx:dx + w_pix, :]
            sl = sl.reshape(tm * w_pix, c_in)
            part = jnp.dot(sl, w_ref[dy * 3 + dx],
                           preferred_element_type=jnp.float32)
            acc = part if acc is None else acc + part
    acc = _apply_act(acc + b_ref[...], act)
    o_ref[0] = acc.reshape(tm, w_pix, -1)


def conv3x3(x, w, b, act=None, tm=40):
    n, h, wd, c = x.shape
    o = w.shape[1]
    assert h % tm == 0, (h, tm)
    t = h // tm
    xp = jnp.pad(x.astype(jnp.bfloat16), ((0, 0), (1, 1), (1, 1), (0, 0)))
    bands = jnp.stack([xp[:, i * tm:i * tm + tm + 2] for i in range(t)], 1)
    bands = bands.reshape(n * t, tm + 2, wd + 2, c)
    kern = functools.partial(_conv3_kernel, tm=tm, w_pix=wd, c_in=c, act=act)
    out = pl.pallas_call(
        kern,
        out_shape=jax.ShapeDtypeStruct((n * t, tm, wd, o), jnp.float32),
        grid=(n * t,),
        in_specs=[
            pl.BlockSpec((1, tm + 2, wd + 2, c), lambda i: (i, 0, 0, 0)),
            pl.BlockSpec((9, c, o), lambda i: (0, 0, 0)),
            pl.BlockSpec((1, o), lambda i: (0, 0)),
        ],
        out_specs=pl.BlockSpec((1, tm, wd, o), lambda i: (i, 0, 0, 0)),
        compiler_params=pltpu.CompilerParams(
            dimension_semantics=("parallel",)),
    )(bands, w.astype(jnp.bfloat16).reshape(9, c, o),
      b.astype(jnp.float32).reshape(1, o))
    return out.reshape(n, h, wd, o)


def _mm_kernel(a_ref, w_ref, b_ref, o_ref, *, act):
    acc = jnp.dot(a_ref[...], w_ref[...], preferred_element_type=jnp.float32)
    o_ref[...] = _apply_act(acc + b_ref[...], act)


def conv_s2(x, w, b, k, pad, act=None, tile=512):
    n, h, wd, c = x.shape
    o = w.shape[1]
    ho = (h + 2 * pad - k) // 2 + 1
    wo = (wd + 2 * pad - k) // 2 + 1
    xp = jnp.pad(x.astype(jnp.bfloat16),
                 ((0, 0), (pad, pad), (pad, pad), (0, 0)))
    taps = [xp[:, dy:dy + 2 * ho:2, dx:dx + 2 * wo:2, :]
            for dy in range(k) for dx in range(k)]
    cols = jnp.concatenate(taps, axis=-1).reshape(n * ho * wo, k * k * c)
    m, kk = cols.shape
    mp = _ceil_to(m, tile)
    if mp != m:
        cols = jnp.pad(cols, ((0, mp - m), (0, 0)))
    kern = functools.partial(_mm_kernel, act=act)
    out = pl.pallas_call(
        kern,
        out_shape=jax.ShapeDtypeStruct((mp, o), jnp.float32),
        grid=(mp // tile,),
        in_specs=[
            pl.BlockSpec((tile, kk), lambda i: (i, 0)),
            pl.BlockSpec((kk, o), lambda i: (0, 0)),
            pl.BlockSpec((1, o), lambda i: (0, 0)),
        ],
        out_specs=pl.BlockSpec((tile, o), lambda i: (i, 0)),
        compiler_params=pltpu.CompilerParams(
            dimension_semantics=("parallel",)),
    )(cols, w.astype(jnp.bfloat16), b.astype(jnp.float32).reshape(1, o))
    return out[:m].reshape(n, ho, wo, o)


def _rate_kernel(y_ref, s_ref, q_ref, bits_ref):
    @pl.when(pl.program_id(0) == 0)
    def _():
        bits_ref[...] = jnp.zeros_like(bits_ref)

    y = y_ref[...]
    q = jnp.round(y)
    q_ref[...] = q
    s = jnp.maximum(s_ref[...], 1e-6)
    p_hi = jax.nn.sigmoid((q + 0.5) / s)
    p_lo = jax.nn.sigmoid((q - 0.5) / s)
    p = jnp.maximum(p_hi - p_lo, 1e-9)
    bits_ref[...] = bits_ref[...] + jnp.sum(-jnp.log(p)) * _INV_LN2


def round_rate(y, scale, tile=512):
    shape = y.shape
    c = shape[-1]
    m = int(np.prod(shape[:-1]))
    assert m % tile == 0, (m, tile)
    y2 = y.reshape(m, c).astype(jnp.float32)
    if scale.ndim == 1:
        s2 = scale.reshape(1, c).astype(jnp.float32)
        s_spec = pl.BlockSpec((1, c), lambda i: (0, 0))
    else:
        s2 = scale.reshape(m, c).astype(jnp.float32)
        s_spec = pl.BlockSpec((tile, c), lambda i: (i, 0))
    q, bits = pl.pallas_call(
        _rate_kernel,
        out_shape=(jax.ShapeDtypeStruct((m, c), jnp.float32),
                   jax.ShapeDtypeStruct((1, 1), jnp.float32)),
        grid=(m // tile,),
        in_specs=[pl.BlockSpec((tile, c), lambda i: (i, 0)), s_spec],
        out_specs=(pl.BlockSpec((tile, c), lambda i: (i, 0)),
                   pl.BlockSpec((1, 1), lambda i: (0, 0))),
        compiler_params=pltpu.CompilerParams(
            dimension_semantics=("arbitrary",)),
    )(y2, s2)
    return q.reshape(shape), bits[0, 0]


def _lerp_kernel(v00, v01, v10, v11, fx, fy, o_ref):
    gx = fx[...]
    gy = fy[...]
    top = v00[...] + gx * (v01[...] - v00[...])
    bot = v10[...] + gx * (v11[...] - v10[...])
    o_ref[...] = top + gy * (bot - top)


def _addclamp_kernel(a_ref, b_ref, o_ref):
    o_ref[...] = jnp.clip(a_ref[...] + b_ref[...], 0.0, 1.0)


def warp(frame, flow):
    n, h, w, c = frame.shape
    gy, gx = jnp.meshgrid(jnp.arange(h, dtype=jnp.float32),
                          jnp.arange(w, dtype=jnp.float32), indexing="ij")
    sx = jnp.clip(gx[None] + flow[..., 0], 0.0, w - 1.0)
    sy = jnp.clip(gy[None] + flow[..., 1], 0.0, h - 1.0)
    x0 = jnp.floor(sx)
    y0 = jnp.floor(sy)
    x1 = jnp.minimum(x0 + 1.0, w - 1.0)
    y1 = jnp.minimum(y0 + 1.0, h - 1.0)
    fx = (sx - x0).reshape(n * h * w, 1)
    fy = (sy - y0).reshape(n * h * w, 1)

    flat = frame.reshape(n, h * w, c)

    def tap(yi, xi):
        idx = (yi.astype(jnp.int32) * w + xi.astype(jnp.int32))
        idx = idx.reshape(n, h * w, 1)
        return jnp.take_along_axis(flat, idx, axis=1).reshape(n * h * w, c)

    m = n * h * w
    tile = 2048
    assert m % tile == 0
    vspec = pl.BlockSpec((tile, c), lambda i: (i, 0))
    fspec = pl.BlockSpec((tile, 1), lambda i: (i, 0))
    out = pl.pallas_call(
        _lerp_kernel,
        out_shape=jax.ShapeDtypeStruct((m, c), jnp.float32),
        grid=(m // tile,),
        in_specs=[vspec, vspec, vspec, vspec, fspec, fspec],
        out_specs=vspec,
        compiler_params=pltpu.CompilerParams(
            dimension_semantics=("parallel",)),
    )(tap(y0, x0), tap(y0, x1), tap(y1, x0), tap(y1, x1), fx, fy)
    return out.reshape(n, h, w, c)


def add_clamp(a, b):
    shape = a.shape
    total = int(np.prod(shape))
    assert total % 128 == 0
    rows = total // 128
    grid = 16
    assert rows % grid == 0 and (rows // grid) % 8 == 0
    tile = rows // grid
    a2 = a.reshape(rows, 128)
    b2 = b.reshape(rows, 128)
    spec = pl.BlockSpec((tile, 128), lambda i: (i, 0))
    out = pl.pallas_call(
        _addclamp_kernel,
        out_shape=jax.ShapeDtypeStruct((rows, 128), jnp.float32),
        grid=(grid,),
        in_specs=[spec, spec],
        out_specs=spec,
        compiler_params=pltpu.CompilerParams(
            dimension_semantics=("parallel",)),
    )(a2, b2)
    return out.reshape(shape)


def _upsample2x_bf16(x):
    x = x.astype(jnp.bfloat16)
    return jnp.repeat(jnp.repeat(x, 2, axis=1), 2, axis=2)


def _encoder(x, c1_w, c1_b, c2_w, c2_b):
    h = conv_s2(x, c1_w, c1_b, k=5, pad=2, act="relu")
    return conv_s2(h, c2_w, c2_b, k=5, pad=2, act=None)


def _decoder(y, c1_w, c1_b, c2_w, c2_b):
    h = conv3x3(_upsample2x_bf16(y), c1_w, c1_b, act="relu")
    return conv3x3(_upsample2x_bf16(h), c2_w, c2_b, act=None)


def _hyperprior(y, henc_w, henc_b, hdec_w, hdec_b, fsr):
    z = conv_s2(jnp.abs(y), henc_w, henc_b, k=3, pad=1, act=None)
    fscale = jax.nn.softplus(fsr)
    z_hat, rate_hyper = round_rate(z, fscale)
    sigma = conv3x3(_upsample2x_bf16(z_hat), hdec_w, hdec_b, act="softplus")
    y_hat, rate_main = round_rate(y, sigma)
    return y_hat, rate_main, rate_hyper


def kernel(me_c1_w, me_c1_b, me_c2_w, me_c2_b,
           mh_henc_w, mh_henc_b, mh_hdec_w, mh_hdec_b, mh_fsr,
           md_c1_w, md_c1_b, md_c2_w, md_c2_b,
           mcn_c1_w, mcn_c1_b, mcn_c2_w, mcn_c2_b, mcn_c3_w, mcn_c3_b,
           re_c1_w, re_c1_b, re_c2_w, re_c2_b,
           rh_henc_w, rh_henc_b, rh_hdec_w, rh_hdec_b, rh_fsr,
           rd_c1_w, rd_c1_b, rd_c2_w, rd_c2_b,
           frame1, frame2, flow12):
    to_nhwc = lambda x: jnp.transpose(x, (0, 2, 3, 1))
    to_nchw = lambda x: jnp.transpose(x, (0, 3, 1, 2))
    f1, f2, fl = to_nhwc(frame1), to_nhwc(frame2), to_nhwc(flow12)

    y_m = _encoder(fl, me_c1_w, me_c1_b, me_c2_w, me_c2_b)
    y_m_hat, rate_motion, rate_hyper_motion = _hyperprior(
        y_m, mh_henc_w, mh_henc_b, mh_hdec_w, mh_hdec_b, mh_fsr)
    flow_rec = _decoder(y_m_hat, md_c1_w, md_c1_b, md_c2_w, md_c2_b)

    warped = warp(f1, flow_rec)
    mc_in = jnp.concatenate([warped, flow_rec, f1], axis=-1)
    hmc = conv3x3(mc_in, mcn_c1_w, mcn_c1_b, act="relu")
    hmc = conv3x3(hmc, mcn_c2_w, mcn_c2_b, act="relu")
    frame2_mc = conv3x3(hmc, mcn_c3_w, mcn_c3_b, act=None)

    residual = f2 - frame2_mc
    y_r = _encoder(residual, re_c1_w, re_c1_b, re_c2_w, re_c2_b)
    y_r_hat, rate_residual, rate_hyper_residual = _hyperprior(
        y_r, rh_henc_w, rh_henc_b, rh_hdec_w, rh_hdec_b, rh_fsr)
    residual_rec = _decoder(y_r_hat, rd_c1_w, rd_c1_b, rd_c2_w, rd_c2_b)

    frame2_rec = add_clamp(frame2_mc, residual_rec)

    return {
        "frame2_reconstructed": to_nchw(frame2_rec),
        "rate_motion": rate_motion,
        "rate_hyper_motion": rate_hyper_motion,
        "rate_residual": rate_residual,
        "rate_hyper_residual": rate_hyper_residual,
        "flow_reconstructed": to_nchw(flow_rec),
        "frame2_motion_compensated": to_nchw(frame2_mc),
        "residual_reconstructed": to_nchw(residual_rec),
    }
